# baseline (speedup 1.0000x reference)
import jax
import jax.numpy as jnp
from jax.experimental import pallas as pl

N = 10000
H1, C1 = 7, 64
H2, C2 = 6, 64
H3, C3 = 6, 40


def _gat_conv(x, edge_index, W, a_src, a_dst, bias, heads, out_ch, num_nodes):
    loops = jnp.arange(num_nodes, dtype=edge_index.dtype)
    src = jnp.concatenate([edge_index[0], loops])
    dst = jnp.concatenate([edge_index[1], loops])
    h = (x @ W).reshape(num_nodes, heads, out_ch)
    alpha_src = (h * a_src[None, :, :]).sum(-1)
    alpha_dst = (h * a_dst[None, :, :]).sum(-1)
    e = alpha_src[src] + alpha_dst[dst]
    e = jax.nn.leaky_relu(e, 0.2)
    e_max = jax.ops.segment_max(e, dst, num_segments=num_nodes)
    ee = jnp.exp(e - e_max[dst])
    den = jax.ops.segment_sum(ee, dst, num_segments=num_nodes)
    alpha = ee / (den[dst] + 1e-16)
    msg = h[src] * alpha[:, :, None]
    out = jax.ops.segment_sum(msg, dst, num_segments=num_nodes)
    return out.reshape(num_nodes, heads * out_ch) + bias


def _bias_add_kernel(x_ref, b_ref, o_ref):
    o_ref[...] = x_ref[...] + b_ref[...]


def _bias_add(x, b):
    return pl.pallas_call(
        _bias_add_kernel,
        out_shape=jax.ShapeDtypeStruct(x.shape, x.dtype),
    )(x, jnp.broadcast_to(b, x.shape))


def kernel(x, edge_index, W1, a1_src, a1_dst, b1, W2, a2_src, a2_dst, b2, W3, a3_src, a3_dst, b3):
    h = _gat_conv(x, edge_index, W1, a1_src, a1_dst, jnp.zeros_like(b1), H1, C1, N)
    h = jax.nn.relu(_bias_add(h, b1))
    h = _gat_conv(h, edge_index, W2, a2_src, a2_dst, jnp.zeros_like(b2), H2, C2, N)
    h = jax.nn.relu(_bias_add(h, b2))
    out = _gat_conv(h, edge_index, W3, a3_src, a3_dst, jnp.zeros_like(b3), H3, C3, N)
    return _bias_add(out, b3)


# speedup vs baseline: 21.8549x; 21.8549x over previous
import dataclasses
import functools

import jax
import jax.numpy as jnp
from jax import lax
from jax.experimental import pallas as pl
from jax.experimental.pallas import tpu as pltpu
from jax.experimental.pallas import tpu_sc as plsc

N = 10000
NP = 10240
IN = 128
E = 320000
NSUB = 32
EPS = 10112
NB = EPS // 128
EPAD = NSUB * EPS
SENT = N
BN = 512
GRID = NP // BN

H1, C1 = 7, 64
H2, C2 = 6, 64
H3, C3 = 6, 40
CP3 = 48

_NEG = -1.0e30


def _t1_body(H, CP, x_ref, w_ref, asm_ref, adm_ref, h_out, as_out, ad_out, m_out):
    i = pl.program_id(0)
    x_blk = x_ref[...]
    h_blk = jnp.dot(x_blk, w_ref[...], preferred_element_type=jnp.float32)
    for h in range(H):
        h_out[h, :, :] = h_blk[:, h * CP:(h + 1) * CP]
    as_blk = jnp.dot(h_blk, asm_ref[...], preferred_element_type=jnp.float32)
    ad_blk = jnp.dot(h_blk, adm_ref[...], preferred_element_type=jnp.float32)
    as_t = as_blk.T
    ad_t = ad_blk.T
    node = i * BN + lax.broadcasted_iota(jnp.int32, (8, BN), 1)
    ad_t = jnp.where(node >= N, _NEG, ad_t)
    as_out[...] = as_t
    ad_out[...] = ad_t
    cat = jnp.concatenate([as_t, ad_t], axis=0)
    bm = jnp.broadcast_to(jnp.max(cat, axis=1, keepdims=True), (16, 128))

    @pl.when(i == 0)
    def _():
        m_out[...] = jnp.full((16, 128), -3.0e38, jnp.float32)

    m_out[...] = jnp.maximum(m_out[...], bm)


def _t1(xp, W, a_src, a_dst, H, CP):
    ind = xp.shape[1]
    hcp = H * CP
    col = lax.broadcasted_iota(jnp.int32, (hcp, 8), 1)
    rowh = lax.broadcasted_iota(jnp.int32, (hcp, 8), 0) // CP
    av_s = jnp.where(col == rowh, a_src.reshape(hcp, 1), 0.0)
    av_d = jnp.where(col == rowh, a_dst.reshape(hcp, 1), 0.0)
    return pl.pallas_call(
        functools.partial(_t1_body, H, CP),
        grid=(GRID,),
        in_specs=[
            pl.BlockSpec((BN, ind), lambda i: (i, 0)),
            pl.BlockSpec((ind, hcp), lambda i: (0, 0)),
            pl.BlockSpec((hcp, 8), lambda i: (0, 0)),
            pl.BlockSpec((hcp, 8), lambda i: (0, 0)),
        ],
        out_specs=[
            pl.BlockSpec((H, BN, CP), lambda i: (0, i, 0)),
            pl.BlockSpec((8, BN), lambda i: (0, i)),
            pl.BlockSpec((8, BN), lambda i: (0, i)),
            pl.BlockSpec((16, 128), lambda i: (0, 0)),
        ],
        out_shape=[
            jax.ShapeDtypeStruct((H, NP, CP), jnp.float32),
            jax.ShapeDtypeStruct((8, NP), jnp.float32),
            jax.ShapeDtypeStruct((8, NP), jnp.float32),
            jax.ShapeDtypeStruct((16, 128), jnp.float32),
        ],
    )(xp, W, av_s, av_d)


def _sc_body(H, CP, h_ref, as_ref, ad_ref, m_ref, src_ref, dst_ref,
             acc_out, den_out,
             asrc_v, adst_v, den_v, src_v, dst_v, rows_v, w_v, m_v,
             acc_s):
    cid = lax.axis_index("c")
    sid = lax.axis_index("s")
    wid = sid * 2 + cid
    slc = NP // 16

    pltpu.sync_copy(src_ref.at[wid], src_v)
    pltpu.sync_copy(dst_ref.at[wid], dst_v)
    pltpu.sync_copy(m_ref, m_v)

    z16 = jnp.zeros((16,), jnp.float32)

    for h in range(H):
        pltpu.sync_copy(as_ref.at[h], asrc_v)
        pltpu.sync_copy(ad_ref.at[h], adst_v)

        @pl.loop(0, NP, step=16)
        def _(i):
            den_v[pl.ds(i, 16)] = z16

        @pl.loop(0, 128)
        def _(r):
            for c in range(CP // 16):
                rows_v[r, pl.ds(c * 16, 16)] = z16

        for t in range(slc // 128):
            pltpu.sync_copy(rows_v, acc_s.at[pl.ds(sid * slc + t * 128, 128)])
        plsc.subcore_barrier()

        mvec = m_v[h, pl.ds(0, 16)] + m_v[8 + h, pl.ds(0, 16)]

        @pl.loop(0, NB)
        def _(j):
            for g in range(8):
                s16 = src_v[j, pl.ds(g * 16, 16)]
                d16 = dst_v[j, pl.ds(g * 16, 16)]
                av = plsc.load_gather(asrc_v, [s16])
                bv = plsc.load_gather(adst_v, [d16])
                e = av + bv
                e = jnp.where(e < 0.0, e * 0.2, e)
                w = jnp.exp(e - mvec)
                plsc.addupdate_scatter(den_v, [d16], w)
                w_v[pl.ds(g * 16, 16)] = w
            pltpu.sync_copy(h_ref.at[h].at[src_v.at[j]], rows_v)

            @pl.loop(0, 128)
            def _(r):
                ws = plsc.load_gather(w_v, [lax.broadcast(r, (16,))])
                for c in range(CP // 16):
                    rows_v[r, pl.ds(c * 16, 16)] = rows_v[r, pl.ds(c * 16, 16)] * ws

            pltpu.sync_copy(rows_v, acc_s.at[dst_v.at[j]], add=True)

        plsc.subcore_barrier()
        pltpu.sync_copy(acc_s.at[pl.ds(sid * slc, slc)],
                        acc_out.at[cid, h, pl.ds(sid * slc, slc)])
        pltpu.sync_copy(den_v, den_out.at[wid, h])


def _sc(h_t, as_t, ad_t, M2, srcp, dstp, H, CP):
    mesh = plsc.VectorSubcoreMesh(core_axis_name="c", subcore_axis_name="s")
    cp = pltpu.CompilerParams(needs_layout_passes=False,
                              use_tc_tiling_on_sc=False)
    fn = pl.kernel(
        functools.partial(_sc_body, H, CP),
        out_type=[
            jax.ShapeDtypeStruct((2, H, NP, CP), jnp.float32),
            jax.ShapeDtypeStruct((NSUB, 8, NP), jnp.float32),
        ],
        mesh=mesh,
        scratch_types=[
            pltpu.VMEM((NP,), jnp.float32),
            pltpu.VMEM((NP,), jnp.float32),
            pltpu.VMEM((NP,), jnp.float32),
            pltpu.VMEM((NB, 128), jnp.int32),
            pltpu.VMEM((NB, 128), jnp.int32),
            pltpu.VMEM((128, CP), jnp.float32),
            pltpu.VMEM((128,), jnp.float32),
            pltpu.VMEM((16, 128), jnp.float32),
            pltpu.VMEM_SHARED((NP, CP), jnp.float32),
        ],
        compiler_params=cp,
    )
    return fn(h_t, as_t, ad_t, M2, srcp, dstp)


def _t2_body(H, CP, CO, relu, acc_ref, den_ref, h_ref, as_ref, ad_ref, m_ref,
             b_ref, o_ref):
    i = pl.program_id(0)
    acc = acc_ref[...]
    den_tot = jnp.sum(den_ref[...], axis=0)
    m = m_ref[...]
    mh = m[0:8, 0:1] + m[8:16, 0:1]
    el = as_ref[...] + ad_ref[...]
    el = jnp.where(el < 0.0, el * 0.2, el)
    wl = jnp.exp(el - mh)
    wl_t = wl.T
    den_t = den_tot.T
    node = i * BN + lax.broadcasted_iota(jnp.int32, (BN, 1), 0)
    live = node < N
    for h in range(H):
        wlh = wl_t[:, h:h + 1]
        num = acc[0, h] + acc[1, h] + wlh * h_ref[h]
        oh = num / (den_t[:, h:h + 1] + wlh + 1e-16)
        oh = oh[:, :CO] + b_ref[:, h * CO:(h + 1) * CO]
        if relu:
            oh = jnp.maximum(oh, 0.0)
        o_ref[:, h * CO:(h + 1) * CO] = jnp.where(live, oh, 0.0)


def _t2(acc, den, h_t, as_t, ad_t, M2, bias, H, CP, CO, relu):
    return pl.pallas_call(
        functools.partial(_t2_body, H, CP, CO, relu),
        grid=(GRID,),
        in_specs=[
            pl.BlockSpec((2, H, BN, CP), lambda i: (0, 0, i, 0)),
            pl.BlockSpec((NSUB, 8, BN), lambda i: (0, 0, i)),
            pl.BlockSpec((H, BN, CP), lambda i: (0, i, 0)),
            pl.BlockSpec((8, BN), lambda i: (0, i)),
            pl.BlockSpec((8, BN), lambda i: (0, i)),
            pl.BlockSpec((16, 128), lambda i: (0, 0)),
            pl.BlockSpec((1, H * CO), lambda i: (0, 0)),
        ],
        out_specs=pl.BlockSpec((BN, H * CO), lambda i: (i, 0)),
        out_shape=jax.ShapeDtypeStruct((NP, H * CO), jnp.float32),
    )(acc, den, h_t, as_t, ad_t, M2, bias.reshape(1, H * CO))


def _layer(xp, edges, W, a_src, a_dst, bias, H, CP, CO, relu):
    srcp, dstp = edges
    h_t, as_t, ad_t, M2 = _t1(xp, W, a_src, a_dst, H, CP)
    acc, den = _sc(h_t, as_t, ad_t, M2, srcp, dstp, H, CP)
    return _t2(acc, den, h_t, as_t, ad_t, M2, bias, H, CP, CO, relu)


def kernel(x, edge_index, W1, a1_src, a1_dst, b1, W2, a2_src, a2_dst, b2,
           W3, a3_src, a3_dst, b3):
    xp = jnp.zeros((NP, IN), jnp.float32).at[:N].set(x)
    ep = jnp.full((2, EPAD), SENT, jnp.int32).at[:, :E].set(edge_index)
    srcp = ep[0].reshape(NSUB, NB, 128)
    dstp = ep[1].reshape(NSUB, NB, 128)
    edges = (srcp, dstp)

    x1 = _layer(xp, edges, W1, a1_src, a1_dst, b1, H1, C1, C1, True)
    x2 = _layer(x1, edges, W2, a2_src, a2_dst, b2, H2, C2, C2, True)

    W3p = jnp.zeros((H2 * C2, H3 * CP3), jnp.float32).reshape(
        H2 * C2, H3, CP3).at[:, :, :C3].set(W3.reshape(H2 * C2, H3, C3)
        ).reshape(H2 * C2, H3 * CP3)
    a3s = jnp.zeros((H3, CP3), jnp.float32).at[:, :C3].set(a3_src)
    a3d = jnp.zeros((H3, CP3), jnp.float32).at[:, :C3].set(a3_dst)
    out = _layer(x2, edges, W3p, a3s, a3d, b3, H3, CP3, C3, False)
    return out[:N]


# speedup vs baseline: 23.0203x; 1.0533x over previous
import dataclasses
import functools

import jax
import jax.numpy as jnp
from jax import lax
from jax.experimental import pallas as pl
from jax.experimental.pallas import tpu as pltpu
from jax.experimental.pallas import tpu_sc as plsc

N = 10000
NP = 10240
IN = 128
E = 320000
NSUB = 32
BSZ = 64
EPS = 10240
NB = EPS // BSZ
EPAD = NSUB * EPS
SENT = N
BN = 512
GRID = NP // BN

H1, C1 = 7, 64
H2, C2 = 6, 64
H3, C3 = 6, 40
CP3 = 48

_NEG = -1.0e30


def _t1_body(H, CP, x_ref, w_ref, asm_ref, adm_ref, h_out, as_out, ad_out, m_out):
    i = pl.program_id(0)
    x_blk = x_ref[...]
    h_blk = jnp.dot(x_blk, w_ref[...], preferred_element_type=jnp.float32)
    for h in range(H):
        h_out[h, :, :] = h_blk[:, h * CP:(h + 1) * CP]
    as_blk = jnp.dot(h_blk, asm_ref[...], preferred_element_type=jnp.float32)
    ad_blk = jnp.dot(h_blk, adm_ref[...], preferred_element_type=jnp.float32)
    as_t = as_blk.T
    ad_t = ad_blk.T
    node = i * BN + lax.broadcasted_iota(jnp.int32, (8, BN), 1)
    ad_t = jnp.where(node >= N, _NEG, ad_t)
    as_out[...] = as_t
    ad_out[...] = ad_t
    cat = jnp.concatenate([as_t, ad_t], axis=0)
    bm = jnp.broadcast_to(jnp.max(cat, axis=1, keepdims=True), (16, 128))

    @pl.when(i == 0)
    def _():
        m_out[...] = jnp.full((16, 128), -3.0e38, jnp.float32)

    m_out[...] = jnp.maximum(m_out[...], bm)


def _t1(xp, W, a_src, a_dst, H, CP):
    ind = xp.shape[1]
    hcp = H * CP
    col = lax.broadcasted_iota(jnp.int32, (hcp, 8), 1)
    rowh = lax.broadcasted_iota(jnp.int32, (hcp, 8), 0) // CP
    av_s = jnp.where(col == rowh, a_src.reshape(hcp, 1), 0.0)
    av_d = jnp.where(col == rowh, a_dst.reshape(hcp, 1), 0.0)
    return pl.pallas_call(
        functools.partial(_t1_body, H, CP),
        grid=(GRID,),
        in_specs=[
            pl.BlockSpec((BN, ind), lambda i: (i, 0)),
            pl.BlockSpec((ind, hcp), lambda i: (0, 0)),
            pl.BlockSpec((hcp, 8), lambda i: (0, 0)),
            pl.BlockSpec((hcp, 8), lambda i: (0, 0)),
        ],
        out_specs=[
            pl.BlockSpec((H, BN, CP), lambda i: (0, i, 0)),
            pl.BlockSpec((8, BN), lambda i: (0, i)),
            pl.BlockSpec((8, BN), lambda i: (0, i)),
            pl.BlockSpec((16, 128), lambda i: (0, 0)),
        ],
        out_shape=[
            jax.ShapeDtypeStruct((H, NP, CP), jnp.float32),
            jax.ShapeDtypeStruct((8, NP), jnp.float32),
            jax.ShapeDtypeStruct((8, NP), jnp.float32),
            jax.ShapeDtypeStruct((16, 128), jnp.float32),
        ],
    )(xp, W, av_s, av_d)


def _sc_body(H, CP, h_ref, as_ref, ad_ref, m_ref, src_ref, dst_ref,
             acc_out, den_out,
             asrc_v, adst_v, den_v, src_v, dst_v,
             r0, r1, r2, r3, w0, w1, w2, w3, m_v,
             sg0, sg1, sg2, sg3, ss0, ss1, ss2, ss3,
             acc_s):
    cid = lax.axis_index("c")
    sid = lax.axis_index("s")
    wid = sid * 2 + cid
    slc = NP // 16
    rows = (r0, r1, r2, r3)
    wv = (w0, w1, w2, w3)
    sg = (sg0, sg1, sg2, sg3)
    ss = (ss0, ss1, ss2, ss3)

    pltpu.sync_copy(src_ref.at[wid], src_v)
    pltpu.sync_copy(dst_ref.at[wid], dst_v)
    pltpu.sync_copy(m_ref, m_v)

    z16 = jnp.zeros((16,), jnp.float32)

    for h in range(H):
        pltpu.sync_copy(as_ref.at[h], asrc_v)
        pltpu.sync_copy(ad_ref.at[h], adst_v)

        @pl.loop(0, NP, step=16)
        def _(i):
            den_v[pl.ds(i, 16)] = z16

        @pl.loop(0, BSZ)
        def _(r):
            for c in range(CP // 16):
                r0[r, pl.ds(c * 16, 16)] = z16

        for t in range(slc // BSZ):
            pltpu.sync_copy(r0, acc_s.at[pl.ds(sid * slc + t * BSZ, BSZ)])
        plsc.subcore_barrier()

        mvec = m_v[h, pl.ds(0, 16)] + m_v[8 + h, pl.ds(0, 16)]

        def alpha_phase(j, wbuf):
            for g in range(BSZ // 16):
                s16 = src_v[j, pl.ds(g * 16, 16)]
                d16 = dst_v[j, pl.ds(g * 16, 16)]
                av = plsc.load_gather(asrc_v, [s16])
                bv = plsc.load_gather(adst_v, [d16])
                e = av + bv
                e = jnp.where(e < 0.0, e * 0.2, e)
                w = jnp.exp(e - mvec)
                plsc.addupdate_scatter(den_v, [d16], w)
                wbuf[pl.ds(g * 16, 16)] = w

        @pl.loop(0, NB // 4)
        def _(k):
            jb = k * 4
            for i in range(4):
                @pl.when(k > 0)
                def _():
                    pltpu.make_async_copy(
                        rows[i], acc_s.at[dst_v.at[jb + i - 4]], ss[i]).wait()
                pltpu.async_copy(h_ref.at[h].at[src_v.at[jb + i]], rows[i],
                                 sg[i])
            for i in range(4):
                alpha_phase(jb + i, wv[i])
            for i in range(4):
                pltpu.make_async_copy(
                    h_ref.at[h].at[src_v.at[jb + i]], rows[i], sg[i]).wait()

                @pl.loop(0, BSZ)
                def _(r):
                    ws = plsc.load_gather(wv[i], [lax.broadcast(r, (16,))])
                    for c in range(CP // 16):
                        rows[i][r, pl.ds(c * 16, 16)] = (
                            rows[i][r, pl.ds(c * 16, 16)] * ws)

                pltpu.async_copy(rows[i], acc_s.at[dst_v.at[jb + i]], ss[i],
                                 add=True)

        for i in range(4):
            pltpu.make_async_copy(
                rows[i], acc_s.at[dst_v.at[NB - 4 + i]], ss[i]).wait()

        plsc.subcore_barrier()
        pltpu.sync_copy(acc_s.at[pl.ds(sid * slc, slc)],
                        acc_out.at[cid, h, pl.ds(sid * slc, slc)])
        pltpu.sync_copy(den_v, den_out.at[wid, h])


def _sc(h_t, as_t, ad_t, M2, srcp, dstp, H, CP):
    mesh = plsc.VectorSubcoreMesh(core_axis_name="c", subcore_axis_name="s")
    cp = pltpu.CompilerParams(needs_layout_passes=False,
                              use_tc_tiling_on_sc=False)
    fn = pl.kernel(
        functools.partial(_sc_body, H, CP),
        out_type=[
            jax.ShapeDtypeStruct((2, H, NP, CP), jnp.float32),
            jax.ShapeDtypeStruct((NSUB, 8, NP), jnp.float32),
        ],
        mesh=mesh,
        scratch_types=[
            pltpu.VMEM((NP,), jnp.float32),
            pltpu.VMEM((NP,), jnp.float32),
            pltpu.VMEM((NP,), jnp.float32),
            pltpu.VMEM((NB, BSZ), jnp.int32),
            pltpu.VMEM((NB, BSZ), jnp.int32),
            pltpu.VMEM((BSZ, CP), jnp.float32),
            pltpu.VMEM((BSZ, CP), jnp.float32),
            pltpu.VMEM((BSZ, CP), jnp.float32),
            pltpu.VMEM((BSZ, CP), jnp.float32),
            pltpu.VMEM((BSZ,), jnp.float32),
            pltpu.VMEM((BSZ,), jnp.float32),
            pltpu.VMEM((BSZ,), jnp.float32),
            pltpu.VMEM((BSZ,), jnp.float32),
            pltpu.VMEM((16, 128), jnp.float32),
            pltpu.SemaphoreType.DMA,
            pltpu.SemaphoreType.DMA,
            pltpu.SemaphoreType.DMA,
            pltpu.SemaphoreType.DMA,
            pltpu.SemaphoreType.DMA,
            pltpu.SemaphoreType.DMA,
            pltpu.SemaphoreType.DMA,
            pltpu.SemaphoreType.DMA,
            pltpu.VMEM_SHARED((NP, CP), jnp.float32),
        ],
        compiler_params=cp,
    )
    return fn(h_t, as_t, ad_t, M2, srcp, dstp)


def _t2_body(H, CP, CO, relu, acc_ref, den_ref, h_ref, as_ref, ad_ref, m_ref,
             b_ref, o_ref):
    i = pl.program_id(0)
    acc = acc_ref[...]
    den_tot = jnp.sum(den_ref[...], axis=0)
    m = m_ref[...]
    mh = m[0:8, 0:1] + m[8:16, 0:1]
    el = as_ref[...] + ad_ref[...]
    el = jnp.where(el < 0.0, el * 0.2, el)
    wl = jnp.exp(el - mh)
    wl_t = wl.T
    den_t = den_tot.T
    node = i * BN + lax.broadcasted_iota(jnp.int32, (BN, 1), 0)
    live = node < N
    for h in range(H):
        wlh = wl_t[:, h:h + 1]
        num = acc[0, h] + acc[1, h] + wlh * h_ref[h]
        oh = num / (den_t[:, h:h + 1] + wlh + 1e-16)
        oh = oh[:, :CO] + b_ref[:, h * CO:(h + 1) * CO]
        if relu:
            oh = jnp.maximum(oh, 0.0)
        o_ref[:, h * CO:(h + 1) * CO] = jnp.where(live, oh, 0.0)


def _t2(acc, den, h_t, as_t, ad_t, M2, bias, H, CP, CO, relu):
    return pl.pallas_call(
        functools.partial(_t2_body, H, CP, CO, relu),
        grid=(GRID,),
        in_specs=[
            pl.BlockSpec((2, H, BN, CP), lambda i: (0, 0, i, 0)),
            pl.BlockSpec((NSUB, 8, BN), lambda i: (0, 0, i)),
            pl.BlockSpec((H, BN, CP), lambda i: (0, i, 0)),
            pl.BlockSpec((8, BN), lambda i: (0, i)),
            pl.BlockSpec((8, BN), lambda i: (0, i)),
            pl.BlockSpec((16, 128), lambda i: (0, 0)),
            pl.BlockSpec((1, H * CO), lambda i: (0, 0)),
        ],
        out_specs=pl.BlockSpec((BN, H * CO), lambda i: (i, 0)),
        out_shape=jax.ShapeDtypeStruct((NP, H * CO), jnp.float32),
    )(acc, den, h_t, as_t, ad_t, M2, bias.reshape(1, H * CO))


def _layer(xp, edges, W, a_src, a_dst, bias, H, CP, CO, relu):
    srcp, dstp = edges
    h_t, as_t, ad_t, M2 = _t1(xp, W, a_src, a_dst, H, CP)
    acc, den = _sc(h_t, as_t, ad_t, M2, srcp, dstp, H, CP)
    return _t2(acc, den, h_t, as_t, ad_t, M2, bias, H, CP, CO, relu)


def kernel(x, edge_index, W1, a1_src, a1_dst, b1, W2, a2_src, a2_dst, b2,
           W3, a3_src, a3_dst, b3):
    xp = jnp.zeros((NP, IN), jnp.float32).at[:N].set(x)
    ep = jnp.full((2, EPAD), SENT, jnp.int32).at[:, :E].set(edge_index)
    srcp = ep[0].reshape(NSUB, NB, BSZ)
    dstp = ep[1].reshape(NSUB, NB, BSZ)
    edges = (srcp, dstp)

    x1 = _layer(xp, edges, W1, a1_src, a1_dst, b1, H1, C1, C1, True)
    x2 = _layer(x1, edges, W2, a2_src, a2_dst, b2, H2, C2, C2, True)

    W3p = jnp.zeros((H2 * C2, H3 * CP3), jnp.float32).reshape(
        H2 * C2, H3, CP3).at[:, :, :C3].set(W3.reshape(H2 * C2, H3, C3)
        ).reshape(H2 * C2, H3 * CP3)
    a3s = jnp.zeros((H3, CP3), jnp.float32).at[:, :C3].set(a3_src)
    a3d = jnp.zeros((H3, CP3), jnp.float32).at[:, :C3].set(a3_dst)
    out = _layer(x2, edges, W3p, a3s, a3d, b3, H3, CP3, C3, False)
    return out[:N]


# speedup vs baseline: 27.0038x; 1.1730x over previous
import dataclasses
import functools

import jax
import jax.numpy as jnp
from jax import lax
from jax.experimental import pallas as pl
from jax.experimental.pallas import tpu as pltpu
from jax.experimental.pallas import tpu_sc as plsc

N = 10000
NP = 10240
IN = 128
E = 320000
NSUB = 32
BSZ = 64
EPS = 10240
NB = EPS // BSZ
EPAD = NSUB * EPS
SENT = N
BN = 512
GRID = NP // BN
CH = 32

H1, C1 = 7, 64
H2, C2 = 6, 64
H3, C3 = 6, 40
CP3 = 64

_NEG = -1.0e30


def _t1_body(H, CP, x_ref, w_ref, asm_ref, adm_ref, h_out, as_out, ad_out, m_out):
    i = pl.program_id(0)
    x_blk = x_ref[...]
    h_blk = jnp.dot(x_blk, w_ref[...], preferred_element_type=jnp.float32)
    for h in range(H):
        for c in range(CP // CH):
            h_out[h, c, :, :] = h_blk[:, h * CP + c * CH:h * CP + (c + 1) * CH]
    as_blk = jnp.dot(h_blk, asm_ref[...], preferred_element_type=jnp.float32)
    ad_blk = jnp.dot(h_blk, adm_ref[...], preferred_element_type=jnp.float32)
    as_t = as_blk.T
    ad_t = ad_blk.T
    node = i * BN + lax.broadcasted_iota(jnp.int32, (8, BN), 1)
    ad_t = jnp.where(node >= N, _NEG, ad_t)
    as_out[...] = as_t
    ad_out[...] = ad_t
    cat = jnp.concatenate([as_t, ad_t], axis=0)
    bm = jnp.broadcast_to(jnp.max(cat, axis=1, keepdims=True), (16, 128))

    @pl.when(i == 0)
    def _():
        m_out[...] = jnp.full((16, 128), -3.0e38, jnp.float32)

    m_out[...] = jnp.maximum(m_out[...], bm)


def _t1(xp, W, a_src, a_dst, H, CP):
    ind = xp.shape[1]
    hcp = H * CP
    nch = CP // CH
    col = lax.broadcasted_iota(jnp.int32, (hcp, 8), 1)
    rowh = lax.broadcasted_iota(jnp.int32, (hcp, 8), 0) // CP
    av_s = jnp.where(col == rowh, a_src.reshape(hcp, 1), 0.0)
    av_d = jnp.where(col == rowh, a_dst.reshape(hcp, 1), 0.0)
    return pl.pallas_call(
        functools.partial(_t1_body, H, CP),
        grid=(GRID,),
        in_specs=[
            pl.BlockSpec((BN, ind), lambda i: (i, 0)),
            pl.BlockSpec((ind, hcp), lambda i: (0, 0)),
            pl.BlockSpec((hcp, 8), lambda i: (0, 0)),
            pl.BlockSpec((hcp, 8), lambda i: (0, 0)),
        ],
        out_specs=[
            pl.BlockSpec((H, nch, BN, CH), lambda i: (0, 0, i, 0)),
            pl.BlockSpec((8, BN), lambda i: (0, i)),
            pl.BlockSpec((8, BN), lambda i: (0, i)),
            pl.BlockSpec((16, 128), lambda i: (0, 0)),
        ],
        out_shape=[
            jax.ShapeDtypeStruct((H, nch, NP, CH), jnp.float32),
            jax.ShapeDtypeStruct((8, NP), jnp.float32),
            jax.ShapeDtypeStruct((8, NP), jnp.float32),
            jax.ShapeDtypeStruct((16, 128), jnp.float32),
        ],
    )(xp, W, av_s, av_d)


def _sc_body(H, CP, h_ref, as_ref, ad_ref, m_ref, src_ref, dst_ref,
             acc_out, den_out,
             asrc_v, adst_v, den_v, src_v, dst_v,
             r0, r1, r2, r3, w0, w1, w2, w3, m_v,
             sg0, sg1, sg2, sg3, ss0, ss1, ss2, ss3,
             h_sh, acc_s):
    cid = lax.axis_index("c")
    sid = lax.axis_index("s")
    wid = sid * 2 + cid
    slc = NP // 16
    nch = CP // CH
    rows = (r0, r1, r2, r3)
    wv = (w0, w1, w2, w3)
    sg = (sg0, sg1, sg2, sg3)
    ss = (ss0, ss1, ss2, ss3)

    pltpu.sync_copy(src_ref.at[wid], src_v)
    pltpu.sync_copy(dst_ref.at[wid], dst_v)
    pltpu.sync_copy(m_ref, m_v)

    z16 = jnp.zeros((16,), jnp.float32)

    for h in range(H):
        pltpu.sync_copy(as_ref.at[h], asrc_v)
        pltpu.sync_copy(ad_ref.at[h], adst_v)
        mvec = m_v[h, pl.ds(0, 16)] + m_v[8 + h, pl.ds(0, 16)]

        for ch in range(nch):
            if ch == 0:
                @pl.loop(0, NP, step=16)
                def _(i):
                    den_v[pl.ds(i, 16)] = z16

            @pl.loop(0, BSZ)
            def _(r):
                for c in range(CH // 16):
                    r0[r, pl.ds(c * 16, 16)] = z16

            for t in range(slc // BSZ):
                pltpu.sync_copy(r0, acc_s.at[pl.ds(sid * slc + t * BSZ, BSZ)])
            pltpu.sync_copy(h_ref.at[h].at[ch].at[pl.ds(sid * slc, slc)],
                            h_sh.at[pl.ds(sid * slc, slc)])
            plsc.subcore_barrier()

            def alpha_phase(j, wbuf):
                for g in range(BSZ // 16):
                    s16 = src_v[j, pl.ds(g * 16, 16)]
                    d16 = dst_v[j, pl.ds(g * 16, 16)]
                    av = plsc.load_gather(asrc_v, [s16])
                    bv = plsc.load_gather(adst_v, [d16])
                    e = av + bv
                    e = jnp.where(e < 0.0, e * 0.2, e)
                    w = jnp.exp(e - mvec)
                    if ch == 0:
                        plsc.addupdate_scatter(den_v, [d16], w)
                    wbuf[pl.ds(g * 16, 16)] = w

            @pl.loop(0, NB // 4)
            def _(k):
                jb = k * 4
                for i in range(4):
                    @pl.when(k > 0)
                    def _():
                        pltpu.make_async_copy(
                            rows[i], acc_s.at[dst_v.at[jb + i - 4]],
                            ss[i]).wait()
                    pltpu.async_copy(h_sh.at[src_v.at[jb + i]], rows[i],
                                     sg[i])
                for i in range(4):
                    alpha_phase(jb + i, wv[i])
                for i in range(4):
                    pltpu.make_async_copy(
                        h_sh.at[src_v.at[jb + i]], rows[i], sg[i]).wait()

                    @pl.loop(0, BSZ)
                    def _(r):
                        ws = plsc.load_gather(wv[i], [lax.broadcast(r, (16,))])
                        for c in range(CH // 16):
                            rows[i][r, pl.ds(c * 16, 16)] = (
                                rows[i][r, pl.ds(c * 16, 16)] * ws)

                    pltpu.async_copy(rows[i], acc_s.at[dst_v.at[jb + i]],
                                     ss[i], add=True)

            for i in range(4):
                pltpu.make_async_copy(
                    rows[i], acc_s.at[dst_v.at[NB - 4 + i]], ss[i]).wait()

            plsc.subcore_barrier()
            pltpu.sync_copy(acc_s.at[pl.ds(sid * slc, slc)],
                            acc_out.at[cid, h, ch, pl.ds(sid * slc, slc)])
            if ch == 0:
                pltpu.sync_copy(den_v, den_out.at[wid, h])


def _sc(h_t, as_t, ad_t, M2, srcp, dstp, H, CP):
    mesh = plsc.VectorSubcoreMesh(core_axis_name="c", subcore_axis_name="s")
    cp = pltpu.CompilerParams(needs_layout_passes=False,
                              use_tc_tiling_on_sc=False)
    nch = CP // CH
    fn = pl.kernel(
        functools.partial(_sc_body, H, CP),
        out_type=[
            jax.ShapeDtypeStruct((2, H, nch, NP, CH), jnp.float32),
            jax.ShapeDtypeStruct((NSUB, 8, NP), jnp.float32),
        ],
        mesh=mesh,
        scratch_types=[
            pltpu.VMEM((NP,), jnp.float32),
            pltpu.VMEM((NP,), jnp.float32),
            pltpu.VMEM((NP,), jnp.float32),
            pltpu.VMEM((NB, BSZ), jnp.int32),
            pltpu.VMEM((NB, BSZ), jnp.int32),
            pltpu.VMEM((BSZ, CH), jnp.float32),
            pltpu.VMEM((BSZ, CH), jnp.float32),
            pltpu.VMEM((BSZ, CH), jnp.float32),
            pltpu.VMEM((BSZ, CH), jnp.float32),
            pltpu.VMEM((BSZ,), jnp.float32),
            pltpu.VMEM((BSZ,), jnp.float32),
            pltpu.VMEM((BSZ,), jnp.float32),
            pltpu.VMEM((BSZ,), jnp.float32),
            pltpu.VMEM((16, 128), jnp.float32),
            pltpu.SemaphoreType.DMA,
            pltpu.SemaphoreType.DMA,
            pltpu.SemaphoreType.DMA,
            pltpu.SemaphoreType.DMA,
            pltpu.SemaphoreType.DMA,
            pltpu.SemaphoreType.DMA,
            pltpu.SemaphoreType.DMA,
            pltpu.SemaphoreType.DMA,
            pltpu.VMEM_SHARED((NP, CH), jnp.float32),
            pltpu.VMEM_SHARED((NP, CH), jnp.float32),
        ],
        compiler_params=cp,
    )
    return fn(h_t, as_t, ad_t, M2, srcp, dstp)


def _t2_body(H, CP, CO, relu, acc_ref, den_ref, h_ref, as_ref, ad_ref, m_ref,
             b_ref, o_ref):
    i = pl.program_id(0)
    nch = CP // CH
    den_tot = jnp.sum(den_ref[...], axis=0)
    m = m_ref[...]
    mh = m[0:8, 0:1] + m[8:16, 0:1]
    el = as_ref[...] + ad_ref[...]
    el = jnp.where(el < 0.0, el * 0.2, el)
    wl = jnp.exp(el - mh)
    wl_t = wl.T
    den_t = den_tot.T
    node = i * BN + lax.broadcasted_iota(jnp.int32, (BN, 1), 0)
    live = node < N
    for h in range(H):
        wlh = wl_t[:, h:h + 1]
        hh = jnp.concatenate([h_ref[h, c] for c in range(nch)], axis=1)
        acch = jnp.concatenate(
            [acc_ref[0, h, c] + acc_ref[1, h, c] for c in range(nch)], axis=1)
        num = acch + wlh * hh
        oh = num / (den_t[:, h:h + 1] + wlh + 1e-16)
        oh = oh[:, :CO] + b_ref[:, h * CO:(h + 1) * CO]
        if relu:
            oh = jnp.maximum(oh, 0.0)
        o_ref[:, h * CO:(h + 1) * CO] = jnp.where(live, oh, 0.0)


def _t2(acc, den, h_t, as_t, ad_t, M2, bias, H, CP, CO, relu):
    nch = CP // CH
    return pl.pallas_call(
        functools.partial(_t2_body, H, CP, CO, relu),
        grid=(GRID,),
        in_specs=[
            pl.BlockSpec((2, H, nch, BN, CH), lambda i: (0, 0, 0, i, 0)),
            pl.BlockSpec((NSUB, 8, BN), lambda i: (0, 0, i)),
            pl.BlockSpec((H, nch, BN, CH), lambda i: (0, 0, i, 0)),
            pl.BlockSpec((8, BN), lambda i: (0, i)),
            pl.BlockSpec((8, BN), lambda i: (0, i)),
            pl.BlockSpec((16, 128), lambda i: (0, 0)),
            pl.BlockSpec((1, H * CO), lambda i: (0, 0)),
        ],
        out_specs=pl.BlockSpec((BN, H * CO), lambda i: (i, 0)),
        out_shape=jax.ShapeDtypeStruct((NP, H * CO), jnp.float32),
    )(acc, den, h_t, as_t, ad_t, M2, bias.reshape(1, H * CO))


def _layer(xp, edges, W, a_src, a_dst, bias, H, CP, CO, relu):
    srcp, dstp = edges
    h_t, as_t, ad_t, M2 = _t1(xp, W, a_src, a_dst, H, CP)
    acc, den = _sc(h_t, as_t, ad_t, M2, srcp, dstp, H, CP)
    return _t2(acc, den, h_t, as_t, ad_t, M2, bias, H, CP, CO, relu)


def kernel(x, edge_index, W1, a1_src, a1_dst, b1, W2, a2_src, a2_dst, b2,
           W3, a3_src, a3_dst, b3):
    xp = jnp.zeros((NP, IN), jnp.float32).at[:N].set(x)
    ep = jnp.full((2, EPAD), SENT, jnp.int32).at[:, :E].set(edge_index)
    srcp = ep[0].reshape(NSUB, NB, BSZ)
    dstp = ep[1].reshape(NSUB, NB, BSZ)
    edges = (srcp, dstp)

    x1 = _layer(xp, edges, W1, a1_src, a1_dst, b1, H1, C1, C1, True)
    x2 = _layer(x1, edges, W2, a2_src, a2_dst, b2, H2, C2, C2, True)

    W3p = jnp.zeros((H2 * C2, H3 * CP3), jnp.float32).reshape(
        H2 * C2, H3, CP3).at[:, :, :C3].set(W3.reshape(H2 * C2, H3, C3)
        ).reshape(H2 * C2, H3 * CP3)
    a3s = jnp.zeros((H3, CP3), jnp.float32).at[:, :C3].set(a3_src)
    a3d = jnp.zeros((H3, CP3), jnp.float32).at[:, :C3].set(a3_dst)
    out = _layer(x2, edges, W3p, a3s, a3d, b3, H3, CP3, C3, False)
    return out[:N]
